# baseline (speedup 1.0000x reference)
import functools

import jax
import jax.numpy as jnp
from jax.experimental import pallas as pl
from jax.experimental.pallas import tpu as pltpu

_N = 4096
_E = 65536
_HID = 256
_K = 5


def _fused_matmul(terms, relu, bm=512):
    M = terms[0][0].shape[0]
    N = terms[0][1].shape[1]
    in_specs = []
    args = []
    for A, W in terms:
        ka = A.shape[1]
        in_specs.append(pl.BlockSpec((bm, ka), lambda i: (i, 0)))
        in_specs.append(pl.BlockSpec((ka, N), lambda i: (0, 0)))
        args += [A, W]
    nt = len(terms)

    def body(*refs):
        out = refs[-1]
        acc = jnp.zeros((bm, N), jnp.float32)
        for t in range(nt):
            acc = acc + jnp.dot(refs[2 * t][...], refs[2 * t + 1][...],
                                preferred_element_type=jnp.float32)
        if relu:
            acc = jnp.maximum(acc, 0.0)
        out[...] = acc

    return pl.pallas_call(
        body,
        grid=(M // bm,),
        in_specs=in_specs,
        out_specs=pl.BlockSpec((bm, N), lambda i: (i, 0)),
        out_shape=jax.ShapeDtypeStruct((M, N), jnp.float32),
    )(*args)


def _mean_agg(h_src, ei, n_dst, w=None):
    msgs = h_src[ei[0]]
    if w is not None:
        msgs = msgs * w[:, None]
    s = jnp.zeros((n_dst, h_src.shape[1]), h_src.dtype).at[ei[1]].add(msgs)
    cnt = jnp.zeros((n_dst,), h_src.dtype).at[ei[1]].add(1.0)
    return s / jnp.maximum(cnt, 1.0)[:, None]


def _embed1(xp, xa, ei_pp, ei_aa, p):
    hp = xp @ p['W_in_p']
    ha = xa @ p['W_in_a']
    for l in range(2):
        sl = str(l)
        agg_p = _mean_agg(hp, ei_pp, _N) @ p['W_pp_' + sl]
        agg_a = _mean_agg(ha, ei_aa, _N) @ p['W_aa_' + sl]
        hp_n = jax.nn.relu(hp @ p['W_self_p_' + sl] + agg_p)
        ha_n = jax.nn.relu(ha @ p['W_self_a_' + sl] + agg_a)
        hp, ha = hp_n, ha_n
    return hp, ha


def _edge_pred0(h, ei, W1, b1, W2, b2):
    e = jnp.concatenate([h[ei[0]], h[ei[1]]], axis=1)
    return (jax.nn.relu(e @ W1 + b1) @ W2 + b2)[:, 0]


def _select(ei, pred0, seed):
    r, c = ei[0], ei[1]
    dm = jnp.zeros((_N, _N), jnp.float32).at[r, c].add(pred0)
    u = jax.random.uniform(jax.random.key(seed), (_N, _N), jnp.float32,
                           1e-6, 1 - 1e-6)
    g = -jnp.log(-jnp.log(u))
    cols = jnp.arange(_N, dtype=jnp.float32)[None, :]
    comb = jnp.where(dm > 0, dm + g, -1e5 - cols)
    _, S = jax.lax.top_k(comb, _K)
    m_row = jnp.sum(dm > 0, axis=1)
    sel = jnp.any(S[r] == c[:, None], axis=1) & (m_row[r] > 0)
    return sel.astype(jnp.float32)


def kernel(x_paper, x_author, ei_pp, ei_aa, ei_pa, ei_ap, batch_paper,
           batch_author, index, params):
    p = params
    hp1, ha1 = _embed1(x_paper, x_author, ei_pp, ei_aa, p)
    pred_pp = _edge_pred0(hp1, ei_pp, p['ep_pp_W1'], p['ep_pp_b1'],
                          p['ep_pp_W2'], p['ep_pp_b2'])
    pred_aa = _edge_pred0(ha1, ei_aa, p['ep_aa_W1'], p['ep_aa_b1'],
                          p['ep_aa_W2'], p['ep_aa_b2'])
    w_pp = _select(ei_pp, pred_pp, 42)
    w_aa = _select(ei_aa, pred_aa, 43)

    hp = _fused_matmul([(x_paper, p['W_in_p'])], relu=False)
    ha = _fused_matmul([(x_author, p['W_in_a'])], relu=False)
    for l in range(2):
        sl = str(l)
        m_pp = _mean_agg(hp, ei_pp, _N, w_pp)
        m_aa = _mean_agg(ha, ei_aa, _N, w_aa)
        m_ap = _mean_agg(ha, ei_ap, _N)
        m_pa = _mean_agg(hp, ei_pa, _N)
        hp_n = _fused_matmul([(hp, p['W_self_p_' + sl]), (m_pp, p['W_pp_' + sl]),
                              (m_ap, p['W_ap_' + sl])], relu=True)
        ha_n = _fused_matmul([(ha, p['W_self_a_' + sl]), (m_aa, p['W_aa_' + sl]),
                              (m_pa, p['W_pa_' + sl])], relu=True)
        hp, ha = hp_n, ha_n

    pool = hp[index].reshape(1, -1)
    y_hat = pool @ p['Wc'] + p['bc']
    return (y_hat, w_pp, w_aa)


# speedup vs baseline: 1.2472x; 1.2472x over previous
import functools

import jax
import jax.numpy as jnp
from jax import lax
from jax.experimental import pallas as pl
from jax.experimental.pallas import tpu as pltpu
from jax.experimental.pallas import tpu_sc as plsc

_N = 4096
_E = 65536
_HID = 256
_K = 5

_NS = 16
_TRASH = 4096
_SLAB_R = 4112


def _sc_agg(h_lo, h_hi, src, dst, wsel=None):
    E = src.shape[0]
    per = E // _NS
    B = 128
    nb = per // B
    mesh = plsc.VectorSubcoreMesh(core_axis_name="c", subcore_axis_name="s")
    src3 = src.reshape(_NS, nb, B)
    dst3 = dst.reshape(_NS, nb, B)
    zeros = jnp.zeros((257, 128), jnp.float32)
    weighted = wsel is not None
    ins = [h_lo, h_hi, src3, dst3] + ([wsel.reshape(_NS, nb, B)] if weighted else []) + [zeros]

    scratch = [
        pltpu.VMEM((nb, B), jnp.int32),
        pltpu.VMEM((nb, B), jnp.int32),
        pltpu.VMEM((nb, B), jnp.float32),
        pltpu.VMEM((B, 128), jnp.float32),
        pltpu.VMEM((257, 128), jnp.float32),
        pltpu.VMEM_SHARED((_SLAB_R, 128), jnp.float32),
        pltpu.SemaphoreType.DMA,
    ]

    def body(*refs):
        if weighted:
            hlo, hhi, srcr, dstr, wr, zr, out, sbuf, dbuf, wbuf, rows, zbuf, slab, sem = refs
        else:
            hlo, hhi, srcr, dstr, zr, out, sbuf, dbuf, wbuf, rows, zbuf, slab, sem = refs
        ci = lax.axis_index("c")
        si = lax.axis_index("s")
        pltpu.sync_copy(zr, zbuf)
        pltpu.sync_copy(zbuf, slab.at[pl.ds(si * 257, 257)])
        pltpu.sync_copy(srcr.at[si], sbuf)
        pltpu.sync_copy(dstr.at[si], dbuf)
        if weighted:
            pltpu.sync_copy(wr.at[si], wbuf)
            for i in range(nb):
                for j in range(B // 16):
                    sl = pl.ds(j * 16, 16)
                    wv = wbuf[i, sl]
                    dv = dbuf[i, sl]
                    dbuf[i, sl] = jnp.where(wv > 0, dv, _TRASH)
        plsc.subcore_barrier()

        def gs(tbl):
            def step(b, carry):
                pltpu.async_copy(tbl.at[sbuf.at[b]], rows, sem).wait()
                pltpu.sync_copy(rows, slab.at[dbuf.at[b]], add=True)
                return carry
            lax.fori_loop(0, nb, step, 0)

        @pl.when(ci == 0)
        def _():
            gs(hlo)

        @pl.when(ci == 1)
        def _():
            gs(hhi)

        plsc.subcore_barrier()
        pltpu.sync_copy(slab.at[pl.ds(si * 256, 256)],
                        out.at[pl.ds(ci * 4096 + si * 256, 256)])

    out = pl.kernel(body, out_type=jax.ShapeDtypeStruct((2 * 4096, 128), jnp.float32),
                    mesh=mesh, scratch_types=scratch)(*ins)
    return out.reshape(2, 4096, 128)


def _sc_counts(d0, d1, d2, d3):
    E = d0.shape[0]
    per = E // _NS
    B = 128
    nb = per // B
    mesh = plsc.VectorSubcoreMesh(core_axis_name="c", subcore_axis_name="s")
    ins = [d.reshape(_NS, nb, B) for d in (d0, d1, d2, d3)]
    ones = jnp.ones((B,), jnp.float32)
    zeros = jnp.zeros((272,), jnp.float32)
    ins += [ones, zeros]

    scratch = [
        pltpu.VMEM((nb, B), jnp.int32),
        pltpu.VMEM((B,), jnp.float32),
        pltpu.VMEM((256,), jnp.float32),
        pltpu.VMEM((272,), jnp.float32),
        pltpu.VMEM_SHARED((2 * 4352,), jnp.float32),
        pltpu.SemaphoreType.DMA,
    ]

    def body(l0, l1, l2, l3, onesr, zr, out, dbuf, ov, cbuf, zbuf, cslab, sem):
        ci = lax.axis_index("c")
        si = lax.axis_index("s")
        pltpu.sync_copy(zr, zbuf)
        pltpu.sync_copy(zbuf, cslab.at[pl.ds(si * 272, 272)])
        pltpu.sync_copy(zbuf, cslab.at[pl.ds(4352 + si * 272, 272)])
        pltpu.sync_copy(onesr, ov)
        plsc.subcore_barrier()

        def count_into(lst, base):
            pltpu.sync_copy(lst.at[si], dbuf)
            for i in range(nb):
                for j in range(B // 16):
                    sl = pl.ds(j * 16, 16)
                    dbuf[i, sl] = dbuf[i, sl] + base

            def step(b, carry):
                pltpu.sync_copy(ov, cslab.at[dbuf.at[b]], add=True)
                return carry
            lax.fori_loop(0, nb, step, 0)

        @pl.when(ci == 0)
        def _():
            count_into(l0, 0)
            count_into(l1, 4352)

        @pl.when(ci == 1)
        def _():
            count_into(l2, 0)
            count_into(l3, 4352)

        plsc.subcore_barrier()
        for k in range(2):
            pltpu.sync_copy(cslab.at[pl.ds(k * 4352 + si * 256, 256)], cbuf)
            for j in range(256 // 16):
                sl = pl.ds(j * 16, 16)
                cbuf[sl] = 1.0 / jnp.maximum(cbuf[sl], 1.0)
            pltpu.sync_copy(cbuf, out.at[pl.ds((2 * ci + k) * 4096 + si * 256, 256)])

    out = pl.kernel(body, out_type=jax.ShapeDtypeStruct((4 * 4096,), jnp.float32),
                    mesh=mesh, scratch_types=scratch)(*ins)
    return out.reshape(4, 4096)


def _fused_matmul(terms, relu, bm=512):
    M = terms[0][0].shape[0]
    N = terms[0][1].shape[1]
    in_specs = []
    args = []
    has_scale = []
    for A, W, s in terms:
        ka = A.shape[1]
        in_specs.append(pl.BlockSpec((bm, ka), lambda i: (i, 0)))
        in_specs.append(pl.BlockSpec((ka, N), lambda i: (0, 0)))
        args += [A, W]
        has_scale.append(s is not None)
        if s is not None:
            in_specs.append(pl.BlockSpec((bm, 1), lambda i: (i, 0)))
            args.append(s)

    def body(*refs):
        out = refs[-1]
        acc = jnp.zeros((bm, N), jnp.float32)
        k = 0
        for t in range(len(terms)):
            a = refs[k][...]
            w = refs[k + 1][...]
            k += 2
            if has_scale[t]:
                a = a * refs[k][...]
                k += 1
            acc = acc + jnp.dot(a, w, preferred_element_type=jnp.float32)
        if relu:
            acc = jnp.maximum(acc, 0.0)
        out[...] = acc

    return pl.pallas_call(
        body,
        grid=(M // bm,),
        in_specs=in_specs,
        out_specs=pl.BlockSpec((bm, N), lambda i: (i, 0)),
        out_shape=jax.ShapeDtypeStruct((M, N), jnp.float32),
    )(*args)


def _mean_agg(h_src, ei, n_dst, w=None):
    msgs = h_src[ei[0]]
    if w is not None:
        msgs = msgs * w[:, None]
    s = jnp.zeros((n_dst, h_src.shape[1]), h_src.dtype).at[ei[1]].add(msgs)
    cnt = jnp.zeros((n_dst,), h_src.dtype).at[ei[1]].add(1.0)
    return s / jnp.maximum(cnt, 1.0)[:, None]


def _embed1(xp, xa, ei_pp, ei_aa, p):
    hp = xp @ p['W_in_p']
    ha = xa @ p['W_in_a']
    for l in range(2):
        sl = str(l)
        agg_p = _mean_agg(hp, ei_pp, _N) @ p['W_pp_' + sl]
        agg_a = _mean_agg(ha, ei_aa, _N) @ p['W_aa_' + sl]
        hp_n = jax.nn.relu(hp @ p['W_self_p_' + sl] + agg_p)
        ha_n = jax.nn.relu(ha @ p['W_self_a_' + sl] + agg_a)
        hp, ha = hp_n, ha_n
    return hp, ha


def _edge_pred0(h, ei, W1, b1, W2, b2):
    e = jnp.concatenate([h[ei[0]], h[ei[1]]], axis=1)
    return (jax.nn.relu(e @ W1 + b1) @ W2 + b2)[:, 0]


def _select(ei, pred0, seed):
    r, c = ei[0], ei[1]
    dm = jnp.zeros((_N, _N), jnp.float32).at[r, c].add(pred0)
    u = jax.random.uniform(jax.random.key(seed), (_N, _N), jnp.float32,
                           1e-6, 1 - 1e-6)
    g = -jnp.log(-jnp.log(u))
    cols = jnp.arange(_N, dtype=jnp.float32)[None, :]
    comb = jnp.where(dm > 0, dm + g, -1e5 - cols)
    _, S = jax.lax.top_k(comb, _K)
    m_row = jnp.sum(dm > 0, axis=1)
    sel = jnp.any(S[r] == c[:, None], axis=1) & (m_row[r] > 0)
    return sel.astype(jnp.float32)


def kernel(x_paper, x_author, ei_pp, ei_aa, ei_pa, ei_ap, batch_paper,
           batch_author, index, params):
    p = params
    hp1, ha1 = _embed1(x_paper, x_author, ei_pp, ei_aa, p)
    pred_pp = _edge_pred0(hp1, ei_pp, p['ep_pp_W1'], p['ep_pp_b1'],
                          p['ep_pp_W2'], p['ep_pp_b2'])
    pred_aa = _edge_pred0(ha1, ei_aa, p['ep_aa_W1'], p['ep_aa_b1'],
                          p['ep_aa_W2'], p['ep_aa_b2'])
    w_pp = _select(ei_pp, pred_pp, 42)
    w_aa = _select(ei_aa, pred_aa, 43)

    inv4 = _sc_counts(ei_pp[1], ei_aa[1], ei_pa[1], ei_ap[1])
    inv_pp = inv4[0].reshape(_N, 1)
    inv_aa = inv4[1].reshape(_N, 1)
    inv_pa = inv4[2].reshape(_N, 1)
    inv_ap = inv4[3].reshape(_N, 1)
    hp = _fused_matmul([(x_paper, p['W_in_p'], None)], relu=False)
    ha = _fused_matmul([(x_author, p['W_in_a'], None)], relu=False)
    for l in range(2):
        sl = str(l)
        hp_lo, hp_hi = hp[:, :128], hp[:, 128:]
        ha_lo, ha_hi = ha[:, :128], ha[:, 128:]
        s_pp = _sc_agg(hp_lo, hp_hi, ei_pp[0], ei_pp[1], w_pp)
        s_aa = _sc_agg(ha_lo, ha_hi, ei_aa[0], ei_aa[1], w_aa)
        s_ap = _sc_agg(ha_lo, ha_hi, ei_ap[0], ei_ap[1])
        s_pa = _sc_agg(hp_lo, hp_hi, ei_pa[0], ei_pa[1])
        W_pp, W_aa = p['W_pp_' + sl], p['W_aa_' + sl]
        W_ap, W_pa = p['W_ap_' + sl], p['W_pa_' + sl]
        hp_n = _fused_matmul(
            [(hp, p['W_self_p_' + sl], None),
             (s_pp[0], W_pp[:128], inv_pp), (s_pp[1], W_pp[128:], inv_pp),
             (s_ap[0], W_ap[:128], inv_ap), (s_ap[1], W_ap[128:], inv_ap)],
            relu=True)
        ha_n = _fused_matmul(
            [(ha, p['W_self_a_' + sl], None),
             (s_aa[0], W_aa[:128], inv_aa), (s_aa[1], W_aa[128:], inv_aa),
             (s_pa[0], W_pa[:128], inv_pa), (s_pa[1], W_pa[128:], inv_pa)],
            relu=True)
        hp, ha = hp_n, ha_n

    pool = hp[index].reshape(1, -1)
    y_hat = pool @ p['Wc'] + p['bc']
    return (y_hat, w_pp, w_aa)


# speedup vs baseline: 1.6579x; 1.3293x over previous
import functools

import jax
import jax.numpy as jnp
from jax import lax
from jax.experimental import pallas as pl
from jax.experimental.pallas import tpu as pltpu
from jax.experimental.pallas import tpu_sc as plsc

_N = 4096
_E = 65536
_HID = 256
_K = 5

_NS = 16
_TRASH = 4096
_SLAB_R = 4112


def _sc_agg(h_lo, h_hi, src, dst, wsel=None):
    E = src.shape[0]
    per = E // _NS
    B = 128
    nb = per // B
    mesh = plsc.VectorSubcoreMesh(core_axis_name="c", subcore_axis_name="s")
    src3 = src.reshape(_NS, nb, B)
    dst3 = dst.reshape(_NS, nb, B)
    zeros = jnp.zeros((257, 128), jnp.float32)
    weighted = wsel is not None
    ins = [h_lo, h_hi, src3, dst3] + ([wsel.reshape(_NS, nb, B)] if weighted else []) + [zeros]

    scratch = [
        pltpu.VMEM((nb, B), jnp.int32),
        pltpu.VMEM((nb, B), jnp.int32),
        pltpu.VMEM((nb, B), jnp.float32),
        pltpu.VMEM((B, 128), jnp.float32),
        pltpu.VMEM((257, 128), jnp.float32),
        pltpu.VMEM_SHARED((_SLAB_R, 128), jnp.float32),
        pltpu.SemaphoreType.DMA,
    ]

    def body(*refs):
        if weighted:
            hlo, hhi, srcr, dstr, wr, zr, out, sbuf, dbuf, wbuf, rows, zbuf, slab, sem = refs
        else:
            hlo, hhi, srcr, dstr, zr, out, sbuf, dbuf, wbuf, rows, zbuf, slab, sem = refs
        ci = lax.axis_index("c")
        si = lax.axis_index("s")
        pltpu.sync_copy(zr, zbuf)
        pltpu.sync_copy(zbuf, slab.at[pl.ds(si * 257, 257)])
        pltpu.sync_copy(srcr.at[si], sbuf)
        pltpu.sync_copy(dstr.at[si], dbuf)
        if weighted:
            pltpu.sync_copy(wr.at[si], wbuf)
            for i in range(nb):
                for j in range(B // 16):
                    sl = pl.ds(j * 16, 16)
                    wv = wbuf[i, sl]
                    dv = dbuf[i, sl]
                    dbuf[i, sl] = jnp.where(wv > 0, dv, _TRASH)
        plsc.subcore_barrier()

        def gs(tbl):
            def step(b, carry):
                pltpu.async_copy(tbl.at[sbuf.at[b]], rows, sem).wait()
                pltpu.sync_copy(rows, slab.at[dbuf.at[b]], add=True)
                return carry
            lax.fori_loop(0, nb, step, 0)

        @pl.when(ci == 0)
        def _():
            gs(hlo)

        @pl.when(ci == 1)
        def _():
            gs(hhi)

        plsc.subcore_barrier()
        pltpu.sync_copy(slab.at[pl.ds(si * 256, 256)],
                        out.at[pl.ds(ci * 4096 + si * 256, 256)])

    out = pl.kernel(body, out_type=jax.ShapeDtypeStruct((2 * 4096, 128), jnp.float32),
                    mesh=mesh, scratch_types=scratch)(*ins)
    return out.reshape(2, 4096, 128)


def _sc_counts(d0, d1, d2, d3):
    E = d0.shape[0]
    per = E // _NS
    B = 128
    nb = per // B
    mesh = plsc.VectorSubcoreMesh(core_axis_name="c", subcore_axis_name="s")
    ins = [d.reshape(_NS, nb, B) for d in (d0, d1, d2, d3)]
    ones = jnp.ones((B,), jnp.float32)
    zeros = jnp.zeros((272,), jnp.float32)
    ins += [ones, zeros]

    scratch = [
        pltpu.VMEM((nb, B), jnp.int32),
        pltpu.VMEM((B,), jnp.float32),
        pltpu.VMEM((256,), jnp.float32),
        pltpu.VMEM((272,), jnp.float32),
        pltpu.VMEM_SHARED((2 * 4352,), jnp.float32),
        pltpu.SemaphoreType.DMA,
    ]

    def body(l0, l1, l2, l3, onesr, zr, out, dbuf, ov, cbuf, zbuf, cslab, sem):
        ci = lax.axis_index("c")
        si = lax.axis_index("s")
        pltpu.sync_copy(zr, zbuf)
        pltpu.sync_copy(zbuf, cslab.at[pl.ds(si * 272, 272)])
        pltpu.sync_copy(zbuf, cslab.at[pl.ds(4352 + si * 272, 272)])
        pltpu.sync_copy(onesr, ov)
        plsc.subcore_barrier()

        def count_into(lst, base):
            pltpu.sync_copy(lst.at[si], dbuf)
            for i in range(nb):
                for j in range(B // 16):
                    sl = pl.ds(j * 16, 16)
                    dbuf[i, sl] = dbuf[i, sl] + base

            def step(b, carry):
                pltpu.sync_copy(ov, cslab.at[dbuf.at[b]], add=True)
                return carry
            lax.fori_loop(0, nb, step, 0)

        @pl.when(ci == 0)
        def _():
            count_into(l0, 0)
            count_into(l1, 4352)

        @pl.when(ci == 1)
        def _():
            count_into(l2, 0)
            count_into(l3, 4352)

        plsc.subcore_barrier()
        for k in range(2):
            pltpu.sync_copy(cslab.at[pl.ds(k * 4352 + si * 256, 256)], cbuf)
            for j in range(256 // 16):
                sl = pl.ds(j * 16, 16)
                cbuf[sl] = 1.0 / jnp.maximum(cbuf[sl], 1.0)
            pltpu.sync_copy(cbuf, out.at[pl.ds((2 * ci + k) * 4096 + si * 256, 256)])

    out = pl.kernel(body, out_type=jax.ShapeDtypeStruct((4 * 4096,), jnp.float32),
                    mesh=mesh, scratch_types=scratch)(*ins)
    return out.reshape(4, 4096)


_RB = 128
_NBLK = _N // _RB
_SLAB_F = _RB * _N
_TRASH_F = _SLAB_F
_NEG = -3.0e38


def _sc_scatter_dg(r, c, pred, g):
    E = r.shape[0]
    per = E // _NS
    B = 128
    nb = per // B
    mesh = plsc.VectorSubcoreMesh(core_axis_name="c", subcore_axis_name="s")
    r3 = r.reshape(_NS, nb, B)
    c3 = c.reshape(_NS, nb, B)
    p3 = pred.reshape(_NS, nb, B)
    g3 = g.reshape(_NS, nb, B)
    zeros = jnp.zeros((8192,), jnp.float32)
    ins = [r3, c3, p3, g3, zeros]

    scratch = [
        pltpu.VMEM((nb, B), jnp.int32),
        pltpu.VMEM((nb, B), jnp.int32),
        pltpu.VMEM((nb, B), jnp.float32),
        pltpu.VMEM((nb, B), jnp.float32),
        pltpu.VMEM((nb, B), jnp.int32),
        pltpu.VMEM((nb, B), jnp.float32),
        pltpu.VMEM((8192,), jnp.float32),
        pltpu.VMEM_SHARED((_SLAB_F + 16,), jnp.float32),
        pltpu.VMEM_SHARED((_SLAB_F + 16,), jnp.float32),
        pltpu.SemaphoreType.DMA,
    ]

    def body(rr, cr, pr, gr, zr, outD, outG,
             fbuf, bbuf, dstage, gstage, ibuf, dsc, zbuf, dslab, gslab, sem):
        ci = lax.axis_index("c")
        si = lax.axis_index("s")
        pltpu.sync_copy(zr, zbuf)
        pltpu.sync_copy(rr.at[si], fbuf)
        pltpu.sync_copy(cr.at[si], bbuf)
        pltpu.sync_copy(pr.at[si], dstage)
        pltpu.sync_copy(gr.at[si], gstage)

        def prep(i, carry):
            for j in range(B // 16):
                sl = pl.ds(j * 16, 16)
                rv = fbuf[i, sl]
                cv = bbuf[i, sl]
                fbuf[i, sl] = (rv & (_RB - 1)) * _N + cv
                bbuf[i, sl] = rv >> 7
            return carry
        lax.fori_loop(0, nb, prep, 0)

        for p in range(_NBLK // 2):
            blk = ci * (_NBLK // 2) + p
            for z in range(4):
                pltpu.sync_copy(zbuf, dslab.at[pl.ds(si * 32768 + z * 8192, 8192)])
            plsc.subcore_barrier()

            def redir(i, carry):
                for j in range(B // 16):
                    sl = pl.ds(j * 16, 16)
                    m = bbuf[i, sl] == blk
                    ibuf[i, sl] = jnp.where(m, fbuf[i, sl], _TRASH_F)
                    dsc[i, sl] = jnp.where(m, dstage[i, sl], 0.0)
                return carry
            lax.fori_loop(0, nb, redir, 0)

            def scat(i, carry):
                pltpu.sync_copy(dsc.at[i], dslab.at[ibuf.at[i]], add=True)
                pltpu.sync_copy(gstage.at[i], gslab.at[ibuf.at[i]])
                return carry
            lax.fori_loop(0, nb, scat, 0)
            plsc.subcore_barrier()
            base = blk * _SLAB_F + si * 32768
            pltpu.sync_copy(dslab.at[pl.ds(si * 32768, 32768)],
                            outD.at[pl.ds(base, 32768)])
            pltpu.sync_copy(gslab.at[pl.ds(si * 32768, 32768)],
                            outG.at[pl.ds(base, 32768)])
            plsc.subcore_barrier()

    return pl.kernel(
        body,
        out_type=[jax.ShapeDtypeStruct((_N * _N,), jnp.float32),
                  jax.ShapeDtypeStruct((_N * _N,), jnp.float32)],
        mesh=mesh, scratch_types=scratch)(*ins)


def _tc_topk(D, G):
    BR = 128

    def body(dref, gref, sref):
        d = dref[...]
        g = gref[...]
        cols = lax.broadcasted_iota(jnp.int32, (BR, _N), 1)
        comb = jnp.where(d > 0, d + g, -1e5 - cols.astype(jnp.float32))
        alive = jnp.sum((d > 0).astype(jnp.int32), axis=1, keepdims=True) > 0
        cols8 = lax.broadcasted_iota(jnp.int32, (BR, 8), 1)
        acc = jnp.full((BR, 8), -1, jnp.int32)
        for k in range(_K):
            mx = jnp.max(comb, axis=1, keepdims=True)
            amx = jnp.min(jnp.where(comb == mx, cols, _N), axis=1, keepdims=True)
            acc = jnp.where((cols8 == k) & alive, amx, acc)
            comb = jnp.where(cols == amx, -1e9, comb)
        sref[...] = acc

    return pl.pallas_call(
        body,
        grid=(_N // BR,),
        in_specs=[pl.BlockSpec((BR, _N), lambda i: (i, 0)),
                  pl.BlockSpec((BR, _N), lambda i: (i, 0))],
        out_specs=pl.BlockSpec((BR, 8), lambda i: (i, 0)),
        out_shape=jax.ShapeDtypeStruct((_N, 8), jnp.int32),
    )(D.reshape(_N, _N), G.reshape(_N, _N))


def _sc_member(r, c, S):
    E = r.shape[0]
    per = E // 32
    B = 128
    nb = per // B
    mesh = plsc.VectorSubcoreMesh(core_axis_name="c", subcore_axis_name="s")
    r3 = r.reshape(32, nb, B)
    c3 = c.reshape(32, nb, B)

    scratch = [
        pltpu.VMEM((nb, B), jnp.int32),
        pltpu.VMEM((nb, B), jnp.int32),
        pltpu.VMEM((nb, B), jnp.int32),
        pltpu.VMEM((nb, B), jnp.int32),
        pltpu.VMEM((per,), jnp.float32),
        pltpu.SemaphoreType.DMA,
    ]

    def body(rr, cr, sr, out, obuf, cbuf, mbuf, abuf, wbuf, sem):
        ci = lax.axis_index("c")
        si = lax.axis_index("s")
        tid = ci * _NS + si
        pltpu.sync_copy(rr.at[tid], obuf)
        pltpu.sync_copy(cr.at[tid], cbuf)

        def toff(i, carry):
            for j in range(B // 16):
                sl = pl.ds(j * 16, 16)
                obuf[i, sl] = obuf[i, sl] * 8
                abuf[i, sl] = jnp.zeros((16,), jnp.int32)
            return carry
        lax.fori_loop(0, nb, toff, 0)

        for j in range(_K):
            def gat(i, carry):
                pltpu.async_copy(sr.at[obuf.at[i]], mbuf.at[i], sem).wait()
                return carry
            lax.fori_loop(0, nb, gat, 0)

            def cmp(i, carry):
                for jj in range(B // 16):
                    sl = pl.ds(jj * 16, 16)
                    eq = mbuf[i, sl] == cbuf[i, sl]
                    abuf[i, sl] = abuf[i, sl] | jnp.where(eq, 1, 0)
                    if j < _K - 1:
                        obuf[i, sl] = obuf[i, sl] + 1
                return carry
            lax.fori_loop(0, nb, cmp, 0)

        def wv(i, carry):
            for jj in range(B // 16):
                sl = pl.ds(jj * 16, 16)
                wbuf[pl.ds(i * B + jj * 16, 16)] = jnp.where(
                    abuf[i, sl] != 0, 1.0, 0.0)
            return carry
        lax.fori_loop(0, nb, wv, 0)
        pltpu.sync_copy(wbuf, out.at[pl.ds(tid * per, per)])

    return pl.kernel(body, out_type=jax.ShapeDtypeStruct((E,), jnp.float32),
                     mesh=mesh, scratch_types=scratch)(r3, c3, S.reshape(_N * 8))


def _gumbel_at(seed, flat_idx):
    x0 = jnp.zeros_like(flat_idx, jnp.uint32)
    x1 = flat_idx.astype(jnp.uint32)
    ks0 = jnp.uint32(0)
    ks1 = jnp.uint32(seed)
    ks2 = jnp.uint32(0 ^ seed ^ 0x1BD11BDA)
    rot = [(13, 15, 26, 6), (17, 29, 16, 24)]

    def rotl(x, d):
        return (x << jnp.uint32(d)) | (x >> jnp.uint32(32 - d))

    x0 = x0 + ks0
    x1 = x1 + ks1
    ks = [ks0, ks1, ks2]
    for i in range(5):
        for rt in rot[i % 2]:
            x0 = x0 + x1
            x1 = rotl(x1, rt)
            x1 = x0 ^ x1
        x0 = x0 + ks[(i + 1) % 3]
        x1 = x1 + ks[(i + 2) % 3] + jnp.uint32(i + 1)
    bits = x0 ^ x1
    fl = lax.bitcast_convert_type((bits >> jnp.uint32(9)) | jnp.uint32(0x3F800000),
                                  jnp.float32) - jnp.float32(1.0)
    span = jnp.float32(1 - 1e-6) - jnp.float32(1e-6)
    u = jnp.maximum(jnp.float32(1e-6), fl * span + jnp.float32(1e-6))
    return -jnp.log(-jnp.log(u))


def _fused_matmul(terms, relu, bm=512):
    M = terms[0][0].shape[0]
    N = terms[0][1].shape[1]
    in_specs = []
    args = []
    has_scale = []
    for A, W, s in terms:
        ka = A.shape[1]
        in_specs.append(pl.BlockSpec((bm, ka), lambda i: (i, 0)))
        in_specs.append(pl.BlockSpec((ka, N), lambda i: (0, 0)))
        args += [A, W]
        has_scale.append(s is not None)
        if s is not None:
            in_specs.append(pl.BlockSpec((bm, 1), lambda i: (i, 0)))
            args.append(s)

    def body(*refs):
        out = refs[-1]
        acc = jnp.zeros((bm, N), jnp.float32)
        k = 0
        for t in range(len(terms)):
            a = refs[k][...]
            w = refs[k + 1][...]
            k += 2
            if has_scale[t]:
                a = a * refs[k][...]
                k += 1
            acc = acc + jnp.dot(a, w, preferred_element_type=jnp.float32)
        if relu:
            acc = jnp.maximum(acc, 0.0)
        out[...] = acc

    return pl.pallas_call(
        body,
        grid=(M // bm,),
        in_specs=in_specs,
        out_specs=pl.BlockSpec((bm, N), lambda i: (i, 0)),
        out_shape=jax.ShapeDtypeStruct((M, N), jnp.float32),
    )(*args)


def _mean_agg(h_src, ei, n_dst, w=None):
    msgs = h_src[ei[0]]
    if w is not None:
        msgs = msgs * w[:, None]
    s = jnp.zeros((n_dst, h_src.shape[1]), h_src.dtype).at[ei[1]].add(msgs)
    cnt = jnp.zeros((n_dst,), h_src.dtype).at[ei[1]].add(1.0)
    return s / jnp.maximum(cnt, 1.0)[:, None]


def _embed1(xp, xa, ei_pp, ei_aa, p):
    hp = xp @ p['W_in_p']
    ha = xa @ p['W_in_a']
    for l in range(2):
        sl = str(l)
        agg_p = _mean_agg(hp, ei_pp, _N) @ p['W_pp_' + sl]
        agg_a = _mean_agg(ha, ei_aa, _N) @ p['W_aa_' + sl]
        hp_n = jax.nn.relu(hp @ p['W_self_p_' + sl] + agg_p)
        ha_n = jax.nn.relu(ha @ p['W_self_a_' + sl] + agg_a)
        hp, ha = hp_n, ha_n
    return hp, ha


def _edge_pred0(h, ei, W1, b1, W2, b2):
    e = jnp.concatenate([h[ei[0]], h[ei[1]]], axis=1)
    return (jax.nn.relu(e @ W1 + b1) @ W2 + b2)[:, 0]


def _select(ei, pred0, seed):
    r, c = ei[0], ei[1]
    g = _gumbel_at(seed, r * _N + c)
    D, G = _sc_scatter_dg(r, c, pred0, g)
    S = _tc_topk(D, G)
    return _sc_member(r, c, S)


def kernel(x_paper, x_author, ei_pp, ei_aa, ei_pa, ei_ap, batch_paper,
           batch_author, index, params):
    p = params
    hp1, ha1 = _embed1(x_paper, x_author, ei_pp, ei_aa, p)
    pred_pp = _edge_pred0(hp1, ei_pp, p['ep_pp_W1'], p['ep_pp_b1'],
                          p['ep_pp_W2'], p['ep_pp_b2'])
    pred_aa = _edge_pred0(ha1, ei_aa, p['ep_aa_W1'], p['ep_aa_b1'],
                          p['ep_aa_W2'], p['ep_aa_b2'])
    w_pp = _select(ei_pp, pred_pp, 42)
    w_aa = _select(ei_aa, pred_aa, 43)

    inv4 = _sc_counts(ei_pp[1], ei_aa[1], ei_pa[1], ei_ap[1])
    inv_pp = inv4[0].reshape(_N, 1)
    inv_aa = inv4[1].reshape(_N, 1)
    inv_pa = inv4[2].reshape(_N, 1)
    inv_ap = inv4[3].reshape(_N, 1)
    hp = _fused_matmul([(x_paper, p['W_in_p'], None)], relu=False)
    ha = _fused_matmul([(x_author, p['W_in_a'], None)], relu=False)
    for l in range(2):
        sl = str(l)
        hp_lo, hp_hi = hp[:, :128], hp[:, 128:]
        ha_lo, ha_hi = ha[:, :128], ha[:, 128:]
        s_pp = _sc_agg(hp_lo, hp_hi, ei_pp[0], ei_pp[1], w_pp)
        s_aa = _sc_agg(ha_lo, ha_hi, ei_aa[0], ei_aa[1], w_aa)
        s_ap = _sc_agg(ha_lo, ha_hi, ei_ap[0], ei_ap[1])
        s_pa = _sc_agg(hp_lo, hp_hi, ei_pa[0], ei_pa[1])
        W_pp, W_aa = p['W_pp_' + sl], p['W_aa_' + sl]
        W_ap, W_pa = p['W_ap_' + sl], p['W_pa_' + sl]
        hp_n = _fused_matmul(
            [(hp, p['W_self_p_' + sl], None),
             (s_pp[0], W_pp[:128], inv_pp), (s_pp[1], W_pp[128:], inv_pp),
             (s_ap[0], W_ap[:128], inv_ap), (s_ap[1], W_ap[128:], inv_ap)],
            relu=True)
        ha_n = _fused_matmul(
            [(ha, p['W_self_a_' + sl], None),
             (s_aa[0], W_aa[:128], inv_aa), (s_aa[1], W_aa[128:], inv_aa),
             (s_pa[0], W_pa[:128], inv_pa), (s_pa[1], W_pa[128:], inv_pa)],
            relu=True)
        hp, ha = hp_n, ha_n

    pool = hp[index].reshape(1, -1)
    y_hat = pool @ p['Wc'] + p['bc']
    return (y_hat, w_pp, w_aa)


# speedup vs baseline: 1.6792x; 1.0128x over previous
import functools

import jax
import jax.numpy as jnp
from jax import lax
from jax.experimental import pallas as pl
from jax.experimental.pallas import tpu as pltpu
from jax.experimental.pallas import tpu_sc as plsc

_N = 4096
_E = 65536
_HID = 256
_K = 5

_NS = 16
_TRASH = 4096
_SLAB_R = 4112


def _sc_agg(h_lo, h_hi, src, dst, wsel=None):
    E = src.shape[0]
    per = E // _NS
    B = 128
    nb = per // B
    mesh = plsc.VectorSubcoreMesh(core_axis_name="c", subcore_axis_name="s")
    src3 = src.reshape(_NS, nb, B)
    dst3 = dst.reshape(_NS, nb, B)
    zeros = jnp.zeros((257, 128), jnp.float32)
    weighted = wsel is not None
    ins = [h_lo, h_hi, src3, dst3] + ([wsel.reshape(_NS, nb, B)] if weighted else []) + [zeros]

    scratch = [
        pltpu.VMEM((nb, B), jnp.int32),
        pltpu.VMEM((nb, B), jnp.int32),
        pltpu.VMEM((nb, B), jnp.float32),
        pltpu.VMEM((B, 128), jnp.float32),
        pltpu.VMEM((257, 128), jnp.float32),
        pltpu.VMEM_SHARED((_SLAB_R, 128), jnp.float32),
        pltpu.SemaphoreType.DMA,
    ]

    def body(*refs):
        if weighted:
            hlo, hhi, srcr, dstr, wr, zr, out, sbuf, dbuf, wbuf, rows, zbuf, slab, sem = refs
        else:
            hlo, hhi, srcr, dstr, zr, out, sbuf, dbuf, wbuf, rows, zbuf, slab, sem = refs
        ci = lax.axis_index("c")
        si = lax.axis_index("s")
        pltpu.sync_copy(zr, zbuf)
        pltpu.sync_copy(zbuf, slab.at[pl.ds(si * 257, 257)])
        pltpu.sync_copy(srcr.at[si], sbuf)
        pltpu.sync_copy(dstr.at[si], dbuf)
        if weighted:
            pltpu.sync_copy(wr.at[si], wbuf)
            for i in range(nb):
                for j in range(B // 16):
                    sl = pl.ds(j * 16, 16)
                    wv = wbuf[i, sl]
                    dv = dbuf[i, sl]
                    dbuf[i, sl] = jnp.where(wv > 0, dv, _TRASH)
        plsc.subcore_barrier()

        def gs(tbl):
            def step(b, carry):
                pltpu.async_copy(tbl.at[sbuf.at[b]], rows, sem).wait()
                pltpu.sync_copy(rows, slab.at[dbuf.at[b]], add=True)
                return carry
            lax.fori_loop(0, nb, step, 0)

        @pl.when(ci == 0)
        def _():
            gs(hlo)

        @pl.when(ci == 1)
        def _():
            gs(hhi)

        plsc.subcore_barrier()
        pltpu.sync_copy(slab.at[pl.ds(si * 256, 256)],
                        out.at[pl.ds(ci * 4096 + si * 256, 256)])

    out = pl.kernel(body, out_type=jax.ShapeDtypeStruct((2 * 4096, 128), jnp.float32),
                    mesh=mesh, scratch_types=scratch)(*ins)
    return out.reshape(2, 4096, 128)


def _sc_counts(d0, d1, d2, d3):
    E = d0.shape[0]
    per = E // _NS
    B = 128
    nb = per // B
    mesh = plsc.VectorSubcoreMesh(core_axis_name="c", subcore_axis_name="s")
    ins = [d.reshape(_NS, nb, B) for d in (d0, d1, d2, d3)]
    ones = jnp.ones((B,), jnp.float32)
    zeros = jnp.zeros((272,), jnp.float32)
    ins += [ones, zeros]

    scratch = [
        pltpu.VMEM((nb, B), jnp.int32),
        pltpu.VMEM((B,), jnp.float32),
        pltpu.VMEM((256,), jnp.float32),
        pltpu.VMEM((272,), jnp.float32),
        pltpu.VMEM_SHARED((2 * 4352,), jnp.float32),
        pltpu.SemaphoreType.DMA,
    ]

    def body(l0, l1, l2, l3, onesr, zr, out, cout, dbuf, ov, cbuf, zbuf, cslab, sem):
        ci = lax.axis_index("c")
        si = lax.axis_index("s")
        pltpu.sync_copy(zr, zbuf)
        pltpu.sync_copy(zbuf, cslab.at[pl.ds(si * 272, 272)])
        pltpu.sync_copy(zbuf, cslab.at[pl.ds(4352 + si * 272, 272)])
        pltpu.sync_copy(onesr, ov)
        plsc.subcore_barrier()

        def count_into(lst, base):
            pltpu.sync_copy(lst.at[si], dbuf)
            for i in range(nb):
                for j in range(B // 16):
                    sl = pl.ds(j * 16, 16)
                    dbuf[i, sl] = dbuf[i, sl] + base

            def step(b, carry):
                pltpu.sync_copy(ov, cslab.at[dbuf.at[b]], add=True)
                return carry
            lax.fori_loop(0, nb, step, 0)

        @pl.when(ci == 0)
        def _():
            count_into(l0, 0)
            count_into(l1, 4352)

        @pl.when(ci == 1)
        def _():
            count_into(l2, 0)
            count_into(l3, 4352)

        plsc.subcore_barrier()
        for k in range(2):
            pltpu.sync_copy(cslab.at[pl.ds(k * 4352 + si * 256, 256)], cbuf)
            pltpu.sync_copy(cbuf, cout.at[pl.ds((2 * ci + k) * 4096 + si * 256, 256)])
            for j in range(256 // 16):
                sl = pl.ds(j * 16, 16)
                cbuf[sl] = 1.0 / jnp.maximum(cbuf[sl], 1.0)
            pltpu.sync_copy(cbuf, out.at[pl.ds((2 * ci + k) * 4096 + si * 256, 256)])

    out, cnt = pl.kernel(body,
                         out_type=[jax.ShapeDtypeStruct((4 * 4096,), jnp.float32),
                                   jax.ShapeDtypeStruct((4 * 4096,), jnp.float32)],
                         mesh=mesh, scratch_types=scratch)(*ins)
    return out.reshape(4, 4096), cnt.reshape(4, 4096)


_RB = 128
_NBLK = _N // _RB
_SLAB_F = _RB * _N
_TRASH_F = _SLAB_F
_NEG = -3.0e38


def _sc_scatter_dg(r, c, pred, g):
    E = r.shape[0]
    per = E // _NS
    B = 128
    nb = per // B
    mesh = plsc.VectorSubcoreMesh(core_axis_name="c", subcore_axis_name="s")
    r3 = r.reshape(_NS, nb, B)
    c3 = c.reshape(_NS, nb, B)
    p3 = pred.reshape(_NS, nb, B)
    g3 = g.reshape(_NS, nb, B)
    zeros = jnp.zeros((8192,), jnp.float32)
    ins = [r3, c3, p3, g3, zeros]

    scratch = [
        pltpu.VMEM((nb, B), jnp.int32),
        pltpu.VMEM((nb, B), jnp.int32),
        pltpu.VMEM((nb, B), jnp.float32),
        pltpu.VMEM((nb, B), jnp.float32),
        pltpu.VMEM((nb, B), jnp.int32),
        pltpu.VMEM((nb, B), jnp.float32),
        pltpu.VMEM((8192,), jnp.float32),
        pltpu.VMEM_SHARED((_SLAB_F + 16,), jnp.float32),
        pltpu.VMEM_SHARED((_SLAB_F + 16,), jnp.float32),
        pltpu.SemaphoreType.DMA,
    ]

    def body(rr, cr, pr, gr, zr, outD, outG,
             fbuf, bbuf, dstage, gstage, ibuf, dsc, zbuf, dslab, gslab, sem):
        ci = lax.axis_index("c")
        si = lax.axis_index("s")
        pltpu.sync_copy(zr, zbuf)
        pltpu.sync_copy(rr.at[si], fbuf)
        pltpu.sync_copy(cr.at[si], bbuf)
        pltpu.sync_copy(pr.at[si], dstage)
        pltpu.sync_copy(gr.at[si], gstage)

        def prep(i, carry):
            for j in range(B // 16):
                sl = pl.ds(j * 16, 16)
                rv = fbuf[i, sl]
                cv = bbuf[i, sl]
                fbuf[i, sl] = (rv & (_RB - 1)) * _N + cv
                bbuf[i, sl] = rv >> 7
            return carry
        lax.fori_loop(0, nb, prep, 0)

        for p in range(_NBLK // 2):
            blk = ci * (_NBLK // 2) + p
            for z in range(4):
                pltpu.sync_copy(zbuf, dslab.at[pl.ds(si * 32768 + z * 8192, 8192)])
            plsc.subcore_barrier()

            def redir(i, carry):
                for j in range(B // 16):
                    sl = pl.ds(j * 16, 16)
                    m = bbuf[i, sl] == blk
                    ibuf[i, sl] = jnp.where(m, fbuf[i, sl], _TRASH_F)
                    dsc[i, sl] = jnp.where(m, dstage[i, sl], 0.0)
                return carry
            lax.fori_loop(0, nb, redir, 0)

            def scat(i, carry):
                pltpu.sync_copy(dsc.at[i], dslab.at[ibuf.at[i]], add=True)
                pltpu.sync_copy(gstage.at[i], gslab.at[ibuf.at[i]])
                return carry
            lax.fori_loop(0, nb, scat, 0)
            plsc.subcore_barrier()
            base = blk * _SLAB_F + si * 32768
            pltpu.sync_copy(dslab.at[pl.ds(si * 32768, 32768)],
                            outD.at[pl.ds(base, 32768)])
            pltpu.sync_copy(gslab.at[pl.ds(si * 32768, 32768)],
                            outG.at[pl.ds(base, 32768)])
            plsc.subcore_barrier()

    return pl.kernel(
        body,
        out_type=[jax.ShapeDtypeStruct((_N * _N,), jnp.float32),
                  jax.ShapeDtypeStruct((_N * _N,), jnp.float32)],
        mesh=mesh, scratch_types=scratch)(*ins)


def _tc_topk(D, G):
    BR = 128

    def body(dref, gref, sref):
        d = dref[...]
        g = gref[...]
        cols = lax.broadcasted_iota(jnp.int32, (BR, _N), 1)
        comb = jnp.where(d > 0, d + g, -1e5 - cols.astype(jnp.float32))
        alive = jnp.sum((d > 0).astype(jnp.int32), axis=1, keepdims=True) > 0
        cols8 = lax.broadcasted_iota(jnp.int32, (BR, 8), 1)
        acc = jnp.full((BR, 8), -1, jnp.int32)
        for k in range(_K):
            mx = jnp.max(comb, axis=1, keepdims=True)
            amx = jnp.min(jnp.where(comb == mx, cols, _N), axis=1, keepdims=True)
            acc = jnp.where((cols8 == k) & alive, amx, acc)
            comb = jnp.where(cols == amx, -1e9, comb)
        sref[...] = acc

    return pl.pallas_call(
        body,
        grid=(_N // BR,),
        in_specs=[pl.BlockSpec((BR, _N), lambda i: (i, 0)),
                  pl.BlockSpec((BR, _N), lambda i: (i, 0))],
        out_specs=pl.BlockSpec((BR, 8), lambda i: (i, 0)),
        out_shape=jax.ShapeDtypeStruct((_N, 8), jnp.int32),
    )(D.reshape(_N, _N), G.reshape(_N, _N))


def _sc_member(r, c, S):
    E = r.shape[0]
    per = E // 32
    B = 128
    nb = per // B
    mesh = plsc.VectorSubcoreMesh(core_axis_name="c", subcore_axis_name="s")
    r3 = r.reshape(32, nb, B)
    c3 = c.reshape(32, nb, B)

    scratch = [
        pltpu.VMEM((nb, B), jnp.int32),
        pltpu.VMEM((nb, B), jnp.int32),
        pltpu.VMEM((nb, B), jnp.int32),
        pltpu.VMEM((nb, B), jnp.int32),
        pltpu.VMEM((per,), jnp.float32),
        pltpu.SemaphoreType.DMA,
    ]

    def body(rr, cr, sr, out, obuf, cbuf, mbuf, abuf, wbuf, sem):
        ci = lax.axis_index("c")
        si = lax.axis_index("s")
        tid = ci * _NS + si
        pltpu.sync_copy(rr.at[tid], obuf)
        pltpu.sync_copy(cr.at[tid], cbuf)

        def toff(i, carry):
            for j in range(B // 16):
                sl = pl.ds(j * 16, 16)
                obuf[i, sl] = obuf[i, sl] * 8
                abuf[i, sl] = jnp.zeros((16,), jnp.int32)
            return carry
        lax.fori_loop(0, nb, toff, 0)

        for j in range(_K):
            def gat(i, carry):
                pltpu.async_copy(sr.at[obuf.at[i]], mbuf.at[i], sem).wait()
                return carry
            lax.fori_loop(0, nb, gat, 0)

            def cmp(i, carry):
                for jj in range(B // 16):
                    sl = pl.ds(jj * 16, 16)
                    eq = mbuf[i, sl] == cbuf[i, sl]
                    abuf[i, sl] = abuf[i, sl] | jnp.where(eq, 1, 0)
                    if j < _K - 1:
                        obuf[i, sl] = obuf[i, sl] + 1
                return carry
            lax.fori_loop(0, nb, cmp, 0)

        def wv(i, carry):
            for jj in range(B // 16):
                sl = pl.ds(jj * 16, 16)
                wbuf[pl.ds(i * B + jj * 16, 16)] = jnp.where(
                    abuf[i, sl] != 0, 1.0, 0.0)
            return carry
        lax.fori_loop(0, nb, wv, 0)
        pltpu.sync_copy(wbuf, out.at[pl.ds(tid * per, per)])

    return pl.kernel(body, out_type=jax.ShapeDtypeStruct((E,), jnp.float32),
                     mesh=mesh, scratch_types=scratch)(r3, c3, S.reshape(_N * 8))


def _gumbel_at(seed, flat_idx):
    x0 = jnp.zeros_like(flat_idx, jnp.uint32)
    x1 = flat_idx.astype(jnp.uint32)
    ks0 = jnp.uint32(0)
    ks1 = jnp.uint32(seed)
    ks2 = jnp.uint32(0 ^ seed ^ 0x1BD11BDA)
    rot = [(13, 15, 26, 6), (17, 29, 16, 24)]

    def rotl(x, d):
        return (x << jnp.uint32(d)) | (x >> jnp.uint32(32 - d))

    x0 = x0 + ks0
    x1 = x1 + ks1
    ks = [ks0, ks1, ks2]
    for i in range(5):
        for rt in rot[i % 2]:
            x0 = x0 + x1
            x1 = rotl(x1, rt)
            x1 = x0 ^ x1
        x0 = x0 + ks[(i + 1) % 3]
        x1 = x1 + ks[(i + 2) % 3] + jnp.uint32(i + 1)
    bits = x0 ^ x1
    fl = lax.bitcast_convert_type((bits >> jnp.uint32(9)) | jnp.uint32(0x3F800000),
                                  jnp.float32) - jnp.float32(1.0)
    span = jnp.float32(1 - 1e-6) - jnp.float32(1e-6)
    u = jnp.maximum(jnp.float32(1e-6), fl * span + jnp.float32(1e-6))
    return -jnp.log(-jnp.log(u))


def _fused_matmul(terms, relu, bm=512):
    M = terms[0][0].shape[0]
    N = terms[0][1].shape[1]
    in_specs = []
    args = []
    has_scale = []
    for A, W, s in terms:
        ka = A.shape[1]
        in_specs.append(pl.BlockSpec((bm, ka), lambda i: (i, 0)))
        in_specs.append(pl.BlockSpec((ka, N), lambda i: (0, 0)))
        args += [A, W]
        has_scale.append(s is not None)
        if s is not None:
            in_specs.append(pl.BlockSpec((bm, 1), lambda i: (i, 0)))
            args.append(s)

    def body(*refs):
        out = refs[-1]
        acc = jnp.zeros((bm, N), jnp.float32)
        k = 0
        for t in range(len(terms)):
            a = refs[k][...]
            w = refs[k + 1][...]
            k += 2
            if has_scale[t]:
                a = a * refs[k][...]
                k += 1
            acc = acc + jnp.dot(a, w, preferred_element_type=jnp.float32)
        if relu:
            acc = jnp.maximum(acc, 0.0)
        out[...] = acc

    return pl.pallas_call(
        body,
        grid=(M // bm,),
        in_specs=in_specs,
        out_specs=pl.BlockSpec((bm, N), lambda i: (i, 0)),
        out_shape=jax.ShapeDtypeStruct((M, N), jnp.float32),
    )(*args)


def _mean_agg_c(h_src, ei, n_dst, cnt):
    msgs = h_src[ei[0]]
    s = jnp.zeros((n_dst, h_src.shape[1]), h_src.dtype).at[ei[1]].add(msgs)
    return s / jnp.maximum(cnt, 1.0)[:, None]


def _mean_agg(h_src, ei, n_dst, w=None):
    msgs = h_src[ei[0]]
    if w is not None:
        msgs = msgs * w[:, None]
    s = jnp.zeros((n_dst, h_src.shape[1]), h_src.dtype).at[ei[1]].add(msgs)
    cnt = jnp.zeros((n_dst,), h_src.dtype).at[ei[1]].add(1.0)
    return s / jnp.maximum(cnt, 1.0)[:, None]


def _embed1(xp, xa, ei_pp, ei_aa, p, cnt_pp, cnt_aa):
    hp = xp @ p['W_in_p']
    ha = xa @ p['W_in_a']
    for l in range(2):
        sl = str(l)
        agg_p = _mean_agg_c(hp, ei_pp, _N, cnt_pp) @ p['W_pp_' + sl]
        agg_a = _mean_agg_c(ha, ei_aa, _N, cnt_aa) @ p['W_aa_' + sl]
        hp_n = jax.nn.relu(hp @ p['W_self_p_' + sl] + agg_p)
        ha_n = jax.nn.relu(ha @ p['W_self_a_' + sl] + agg_a)
        hp, ha = hp_n, ha_n
    return hp, ha


def _edge_pred0(h, ei, W1, b1, W2, b2):
    e = jnp.concatenate([h[ei[0]], h[ei[1]]], axis=1)
    return (jax.nn.relu(e @ W1 + b1) @ W2 + b2)[:, 0]


def _select(ei, pred0, seed):
    r, c = ei[0], ei[1]
    g = _gumbel_at(seed, r * _N + c)
    D, G = _sc_scatter_dg(r, c, pred0, g)
    S = _tc_topk(D, G)
    return _sc_member(r, c, S)


def kernel(x_paper, x_author, ei_pp, ei_aa, ei_pa, ei_ap, batch_paper,
           batch_author, index, params):
    p = params
    inv4, cnt4 = _sc_counts(ei_pp[1], ei_aa[1], ei_pa[1], ei_ap[1])
    hp1, ha1 = _embed1(x_paper, x_author, ei_pp, ei_aa, p, cnt4[0], cnt4[1])
    pred_pp = _edge_pred0(hp1, ei_pp, p['ep_pp_W1'], p['ep_pp_b1'],
                          p['ep_pp_W2'], p['ep_pp_b2'])
    pred_aa = _edge_pred0(ha1, ei_aa, p['ep_aa_W1'], p['ep_aa_b1'],
                          p['ep_aa_W2'], p['ep_aa_b2'])
    w_pp = _select(ei_pp, pred_pp, 42)
    w_aa = _select(ei_aa, pred_aa, 43)

    inv_pp = inv4[0].reshape(_N, 1)
    inv_aa = inv4[1].reshape(_N, 1)
    inv_pa = inv4[2].reshape(_N, 1)
    inv_ap = inv4[3].reshape(_N, 1)
    hp = _fused_matmul([(x_paper, p['W_in_p'], None)], relu=False)
    ha = _fused_matmul([(x_author, p['W_in_a'], None)], relu=False)
    for l in range(2):
        sl = str(l)
        hp_lo, hp_hi = hp[:, :128], hp[:, 128:]
        ha_lo, ha_hi = ha[:, :128], ha[:, 128:]
        s_pp = _sc_agg(hp_lo, hp_hi, ei_pp[0], ei_pp[1], w_pp)
        s_aa = _sc_agg(ha_lo, ha_hi, ei_aa[0], ei_aa[1], w_aa)
        s_ap = _sc_agg(ha_lo, ha_hi, ei_ap[0], ei_ap[1])
        s_pa = _sc_agg(hp_lo, hp_hi, ei_pa[0], ei_pa[1])
        W_pp, W_aa = p['W_pp_' + sl], p['W_aa_' + sl]
        W_ap, W_pa = p['W_ap_' + sl], p['W_pa_' + sl]
        hp_n = _fused_matmul(
            [(hp, p['W_self_p_' + sl], None),
             (s_pp[0], W_pp[:128], inv_pp), (s_pp[1], W_pp[128:], inv_pp),
             (s_ap[0], W_ap[:128], inv_ap), (s_ap[1], W_ap[128:], inv_ap)],
            relu=True)
        ha_n = _fused_matmul(
            [(ha, p['W_self_a_' + sl], None),
             (s_aa[0], W_aa[:128], inv_aa), (s_aa[1], W_aa[128:], inv_aa),
             (s_pa[0], W_pa[:128], inv_pa), (s_pa[1], W_pa[128:], inv_pa)],
            relu=True)
        hp, ha = hp_n, ha_n

    pool = hp[index].reshape(1, -1)
    y_hat = pool @ p['Wc'] + p['bc']
    return (y_hat, w_pp, w_aa)


# speedup vs baseline: 1.7013x; 1.0131x over previous
import functools

import jax
import jax.numpy as jnp
from jax import lax
from jax.experimental import pallas as pl
from jax.experimental.pallas import tpu as pltpu
from jax.experimental.pallas import tpu_sc as plsc

_N = 4096
_E = 65536
_HID = 256
_K = 5

_NS = 16
_TRASH = 4096
_SLAB_R = 4112


def _sc_agg(h_lo, h_hi, src, dst, wsel=None):
    E = src.shape[0]
    per = E // _NS
    B = 128
    nb = per // B
    mesh = plsc.VectorSubcoreMesh(core_axis_name="c", subcore_axis_name="s")
    src3 = src.reshape(_NS, nb, B)
    dst3 = dst.reshape(_NS, nb, B)
    zeros = jnp.zeros((257, 128), jnp.float32)
    weighted = wsel is not None
    ins = [h_lo, h_hi, src3, dst3] + ([wsel.reshape(_NS, nb, B)] if weighted else []) + [zeros]

    scratch = [
        pltpu.VMEM((nb, B), jnp.int32),
        pltpu.VMEM((nb, B), jnp.int32),
        pltpu.VMEM((nb, B), jnp.float32),
        pltpu.VMEM((B, 128), jnp.float32),
        pltpu.VMEM((B, 128), jnp.float32),
        pltpu.VMEM((257, 128), jnp.float32),
        pltpu.VMEM_SHARED((_SLAB_R, 128), jnp.float32),
        pltpu.SemaphoreType.DMA,
        pltpu.SemaphoreType.DMA,
    ]

    def body(*refs):
        if weighted:
            hlo, hhi, srcr, dstr, wr, zr, out, sbuf, dbuf, wbuf, rows, rows1, zbuf, slab, sem, sem1 = refs
        else:
            hlo, hhi, srcr, dstr, zr, out, sbuf, dbuf, wbuf, rows, rows1, zbuf, slab, sem, sem1 = refs
        ci = lax.axis_index("c")
        si = lax.axis_index("s")
        pltpu.sync_copy(zr, zbuf)
        pltpu.sync_copy(zbuf, slab.at[pl.ds(si * 257, 257)])
        pltpu.sync_copy(srcr.at[si], sbuf)
        pltpu.sync_copy(dstr.at[si], dbuf)
        if weighted:
            pltpu.sync_copy(wr.at[si], wbuf)
            for i in range(nb):
                for j in range(B // 16):
                    sl = pl.ds(j * 16, 16)
                    wv = wbuf[i, sl]
                    dv = dbuf[i, sl]
                    dbuf[i, sl] = jnp.where(wv > 0, dv, _TRASH)
        plsc.subcore_barrier()

        def gs(tbl):
            pltpu.async_copy(tbl.at[sbuf.at[0]], rows, sem)

            def step(i, carry):
                b0 = 2 * i
                pltpu.make_async_copy(tbl.at[sbuf.at[b0]], rows, sem).wait()
                pltpu.async_copy(tbl.at[sbuf.at[(b0 + 1) % nb]], rows1, sem1)
                pltpu.sync_copy(rows, slab.at[dbuf.at[b0]], add=True)
                pltpu.make_async_copy(tbl.at[sbuf.at[b0]], rows1, sem1).wait()
                pltpu.async_copy(tbl.at[sbuf.at[(b0 + 2) % nb]], rows, sem)
                pltpu.sync_copy(rows1, slab.at[dbuf.at[(b0 + 1) % nb]], add=True)
                return carry
            lax.fori_loop(0, nb // 2, step, 0)
            pltpu.make_async_copy(tbl.at[sbuf.at[0]], rows, sem).wait()

        @pl.when(ci == 0)
        def _():
            gs(hlo)

        @pl.when(ci == 1)
        def _():
            gs(hhi)

        plsc.subcore_barrier()
        pltpu.sync_copy(slab.at[pl.ds(si * 256, 256)],
                        out.at[pl.ds(ci * 4096 + si * 256, 256)])

    out = pl.kernel(body, out_type=jax.ShapeDtypeStruct((2 * 4096, 128), jnp.float32),
                    mesh=mesh, scratch_types=scratch)(*ins)
    return out.reshape(2, 4096, 128)


def _sc_counts(d0, d1, d2, d3):
    E = d0.shape[0]
    per = E // _NS
    B = 128
    nb = per // B
    mesh = plsc.VectorSubcoreMesh(core_axis_name="c", subcore_axis_name="s")
    ins = [d.reshape(_NS, nb, B) for d in (d0, d1, d2, d3)]
    ones = jnp.ones((B,), jnp.float32)
    zeros = jnp.zeros((272,), jnp.float32)
    ins += [ones, zeros]

    scratch = [
        pltpu.VMEM((nb, B), jnp.int32),
        pltpu.VMEM((B,), jnp.float32),
        pltpu.VMEM((256,), jnp.float32),
        pltpu.VMEM((272,), jnp.float32),
        pltpu.VMEM_SHARED((2 * 4352,), jnp.float32),
        pltpu.SemaphoreType.DMA,
    ]

    def body(l0, l1, l2, l3, onesr, zr, out, cout, dbuf, ov, cbuf, zbuf, cslab, sem):
        ci = lax.axis_index("c")
        si = lax.axis_index("s")
        pltpu.sync_copy(zr, zbuf)
        pltpu.sync_copy(zbuf, cslab.at[pl.ds(si * 272, 272)])
        pltpu.sync_copy(zbuf, cslab.at[pl.ds(4352 + si * 272, 272)])
        pltpu.sync_copy(onesr, ov)
        plsc.subcore_barrier()

        def count_into(lst, base):
            pltpu.sync_copy(lst.at[si], dbuf)
            for i in range(nb):
                for j in range(B // 16):
                    sl = pl.ds(j * 16, 16)
                    dbuf[i, sl] = dbuf[i, sl] + base

            def step(b, carry):
                pltpu.sync_copy(ov, cslab.at[dbuf.at[b]], add=True)
                return carry
            lax.fori_loop(0, nb, step, 0)

        @pl.when(ci == 0)
        def _():
            count_into(l0, 0)
            count_into(l1, 4352)

        @pl.when(ci == 1)
        def _():
            count_into(l2, 0)
            count_into(l3, 4352)

        plsc.subcore_barrier()
        for k in range(2):
            pltpu.sync_copy(cslab.at[pl.ds(k * 4352 + si * 256, 256)], cbuf)
            pltpu.sync_copy(cbuf, cout.at[pl.ds((2 * ci + k) * 4096 + si * 256, 256)])
            for j in range(256 // 16):
                sl = pl.ds(j * 16, 16)
                cbuf[sl] = 1.0 / jnp.maximum(cbuf[sl], 1.0)
            pltpu.sync_copy(cbuf, out.at[pl.ds((2 * ci + k) * 4096 + si * 256, 256)])

    out, cnt = pl.kernel(body,
                         out_type=[jax.ShapeDtypeStruct((4 * 4096,), jnp.float32),
                                   jax.ShapeDtypeStruct((4 * 4096,), jnp.float32)],
                         mesh=mesh, scratch_types=scratch)(*ins)
    return out.reshape(4, 4096), cnt.reshape(4, 4096)


_RB = 128
_NBLK = _N // _RB
_SLAB_F = _RB * _N
_TRASH_F = _SLAB_F
_NEG = -3.0e38


def _sc_scatter_dg(r, c, pred, g):
    E = r.shape[0]
    per = E // _NS
    B = 128
    nb = per // B
    mesh = plsc.VectorSubcoreMesh(core_axis_name="c", subcore_axis_name="s")
    r3 = r.reshape(_NS, nb, B)
    c3 = c.reshape(_NS, nb, B)
    p3 = pred.reshape(_NS, nb, B)
    g3 = g.reshape(_NS, nb, B)
    zeros = jnp.zeros((8192,), jnp.float32)
    ins = [r3, c3, p3, g3, zeros]

    scratch = [
        pltpu.VMEM((nb, B), jnp.int32),
        pltpu.VMEM((nb, B), jnp.int32),
        pltpu.VMEM((nb, B), jnp.float32),
        pltpu.VMEM((nb, B), jnp.float32),
        pltpu.VMEM((nb, B), jnp.int32),
        pltpu.VMEM((nb, B), jnp.float32),
        pltpu.VMEM((8192,), jnp.float32),
        pltpu.VMEM_SHARED((_SLAB_F + 16,), jnp.float32),
        pltpu.VMEM_SHARED((_SLAB_F + 16,), jnp.float32),
        pltpu.SemaphoreType.DMA,
    ]

    def body(rr, cr, pr, gr, zr, outD, outG,
             fbuf, bbuf, dstage, gstage, ibuf, dsc, zbuf, dslab, gslab, sem):
        ci = lax.axis_index("c")
        si = lax.axis_index("s")
        pltpu.sync_copy(zr, zbuf)
        pltpu.sync_copy(rr.at[si], fbuf)
        pltpu.sync_copy(cr.at[si], bbuf)
        pltpu.sync_copy(pr.at[si], dstage)
        pltpu.sync_copy(gr.at[si], gstage)

        def prep(i, carry):
            for j in range(B // 16):
                sl = pl.ds(j * 16, 16)
                rv = fbuf[i, sl]
                cv = bbuf[i, sl]
                fbuf[i, sl] = (rv & (_RB - 1)) * _N + cv
                bbuf[i, sl] = rv >> 7
            return carry
        lax.fori_loop(0, nb, prep, 0)

        for p in range(_NBLK // 2):
            blk = ci * (_NBLK // 2) + p
            for z in range(4):
                pltpu.sync_copy(zbuf, dslab.at[pl.ds(si * 32768 + z * 8192, 8192)])
            plsc.subcore_barrier()

            def redir(i, carry):
                for j in range(B // 16):
                    sl = pl.ds(j * 16, 16)
                    m = bbuf[i, sl] == blk
                    ibuf[i, sl] = jnp.where(m, fbuf[i, sl], _TRASH_F)
                    dsc[i, sl] = jnp.where(m, dstage[i, sl], 0.0)
                return carry
            lax.fori_loop(0, nb, redir, 0)

            def scat(i, carry):
                pltpu.sync_copy(dsc.at[i], dslab.at[ibuf.at[i]], add=True)
                pltpu.sync_copy(gstage.at[i], gslab.at[ibuf.at[i]])
                return carry
            lax.fori_loop(0, nb, scat, 0)
            plsc.subcore_barrier()
            base = blk * _SLAB_F + si * 32768
            pltpu.sync_copy(dslab.at[pl.ds(si * 32768, 32768)],
                            outD.at[pl.ds(base, 32768)])
            pltpu.sync_copy(gslab.at[pl.ds(si * 32768, 32768)],
                            outG.at[pl.ds(base, 32768)])
            plsc.subcore_barrier()

    return pl.kernel(
        body,
        out_type=[jax.ShapeDtypeStruct((_N * _N,), jnp.float32),
                  jax.ShapeDtypeStruct((_N * _N,), jnp.float32)],
        mesh=mesh, scratch_types=scratch)(*ins)


def _tc_topk(D, G):
    BR = 128

    def body(dref, gref, sref):
        d = dref[...]
        g = gref[...]
        cols = lax.broadcasted_iota(jnp.int32, (BR, _N), 1)
        comb = jnp.where(d > 0, d + g, -1e5 - cols.astype(jnp.float32))
        alive = jnp.sum((d > 0).astype(jnp.int32), axis=1, keepdims=True) > 0
        cols8 = lax.broadcasted_iota(jnp.int32, (BR, 8), 1)
        acc = jnp.full((BR, 8), -1, jnp.int32)
        for k in range(_K):
            mx = jnp.max(comb, axis=1, keepdims=True)
            amx = jnp.min(jnp.where(comb == mx, cols, _N), axis=1, keepdims=True)
            acc = jnp.where((cols8 == k) & alive, amx, acc)
            comb = jnp.where(cols == amx, -1e9, comb)
        sref[...] = acc

    return pl.pallas_call(
        body,
        grid=(_N // BR,),
        in_specs=[pl.BlockSpec((BR, _N), lambda i: (i, 0)),
                  pl.BlockSpec((BR, _N), lambda i: (i, 0))],
        out_specs=pl.BlockSpec((BR, 8), lambda i: (i, 0)),
        out_shape=jax.ShapeDtypeStruct((_N, 8), jnp.int32),
    )(D.reshape(_N, _N), G.reshape(_N, _N))


def _sc_member(r, c, S):
    E = r.shape[0]
    per = E // 32
    B = 128
    nb = per // B
    mesh = plsc.VectorSubcoreMesh(core_axis_name="c", subcore_axis_name="s")
    r3 = r.reshape(32, nb, B)
    c3 = c.reshape(32, nb, B)

    scratch = [
        pltpu.VMEM((nb, B), jnp.int32),
        pltpu.VMEM((nb, B), jnp.int32),
        pltpu.VMEM((nb, B), jnp.int32),
        pltpu.VMEM((nb, B), jnp.int32),
        pltpu.VMEM((per,), jnp.float32),
        pltpu.SemaphoreType.DMA,
    ]

    def body(rr, cr, sr, out, obuf, cbuf, mbuf, abuf, wbuf, sem):
        ci = lax.axis_index("c")
        si = lax.axis_index("s")
        tid = ci * _NS + si
        pltpu.sync_copy(rr.at[tid], obuf)
        pltpu.sync_copy(cr.at[tid], cbuf)

        def toff(i, carry):
            for j in range(B // 16):
                sl = pl.ds(j * 16, 16)
                obuf[i, sl] = obuf[i, sl] * 8
                abuf[i, sl] = jnp.zeros((16,), jnp.int32)
            return carry
        lax.fori_loop(0, nb, toff, 0)

        for j in range(_K):
            def gat(i, carry):
                pltpu.async_copy(sr.at[obuf.at[i]], mbuf.at[i], sem).wait()
                return carry
            lax.fori_loop(0, nb, gat, 0)

            def cmp(i, carry):
                for jj in range(B // 16):
                    sl = pl.ds(jj * 16, 16)
                    eq = mbuf[i, sl] == cbuf[i, sl]
                    abuf[i, sl] = abuf[i, sl] | jnp.where(eq, 1, 0)
                    if j < _K - 1:
                        obuf[i, sl] = obuf[i, sl] + 1
                return carry
            lax.fori_loop(0, nb, cmp, 0)

        def wv(i, carry):
            for jj in range(B // 16):
                sl = pl.ds(jj * 16, 16)
                wbuf[pl.ds(i * B + jj * 16, 16)] = jnp.where(
                    abuf[i, sl] != 0, 1.0, 0.0)
            return carry
        lax.fori_loop(0, nb, wv, 0)
        pltpu.sync_copy(wbuf, out.at[pl.ds(tid * per, per)])

    return pl.kernel(body, out_type=jax.ShapeDtypeStruct((E,), jnp.float32),
                     mesh=mesh, scratch_types=scratch)(r3, c3, S.reshape(_N * 8))


def _gumbel_at(seed, flat_idx):
    x0 = jnp.zeros_like(flat_idx, jnp.uint32)
    x1 = flat_idx.astype(jnp.uint32)
    ks0 = jnp.uint32(0)
    ks1 = jnp.uint32(seed)
    ks2 = jnp.uint32(0 ^ seed ^ 0x1BD11BDA)
    rot = [(13, 15, 26, 6), (17, 29, 16, 24)]

    def rotl(x, d):
        return (x << jnp.uint32(d)) | (x >> jnp.uint32(32 - d))

    x0 = x0 + ks0
    x1 = x1 + ks1
    ks = [ks0, ks1, ks2]
    for i in range(5):
        for rt in rot[i % 2]:
            x0 = x0 + x1
            x1 = rotl(x1, rt)
            x1 = x0 ^ x1
        x0 = x0 + ks[(i + 1) % 3]
        x1 = x1 + ks[(i + 2) % 3] + jnp.uint32(i + 1)
    bits = x0 ^ x1
    fl = lax.bitcast_convert_type((bits >> jnp.uint32(9)) | jnp.uint32(0x3F800000),
                                  jnp.float32) - jnp.float32(1.0)
    span = jnp.float32(1 - 1e-6) - jnp.float32(1e-6)
    u = jnp.maximum(jnp.float32(1e-6), fl * span + jnp.float32(1e-6))
    return -jnp.log(-jnp.log(u))


def _fused_matmul(terms, relu, bm=512):
    M = terms[0][0].shape[0]
    N = terms[0][1].shape[1]
    in_specs = []
    args = []
    has_scale = []
    for A, W, s in terms:
        ka = A.shape[1]
        in_specs.append(pl.BlockSpec((bm, ka), lambda i: (i, 0)))
        in_specs.append(pl.BlockSpec((ka, N), lambda i: (0, 0)))
        args += [A, W]
        has_scale.append(s is not None)
        if s is not None:
            in_specs.append(pl.BlockSpec((bm, 1), lambda i: (i, 0)))
            args.append(s)

    def body(*refs):
        out = refs[-1]
        acc = jnp.zeros((bm, N), jnp.float32)
        k = 0
        for t in range(len(terms)):
            a = refs[k][...]
            w = refs[k + 1][...]
            k += 2
            if has_scale[t]:
                a = a * refs[k][...]
                k += 1
            acc = acc + jnp.dot(a, w, preferred_element_type=jnp.float32)
        if relu:
            acc = jnp.maximum(acc, 0.0)
        out[...] = acc

    return pl.pallas_call(
        body,
        grid=(M // bm,),
        in_specs=in_specs,
        out_specs=pl.BlockSpec((bm, N), lambda i: (i, 0)),
        out_shape=jax.ShapeDtypeStruct((M, N), jnp.float32),
    )(*args)


def _mean_agg_c(h_src, ei, n_dst, cnt):
    msgs = h_src[ei[0]]
    s = jnp.zeros((n_dst, h_src.shape[1]), h_src.dtype).at[ei[1]].add(msgs)
    return s / jnp.maximum(cnt, 1.0)[:, None]


def _mean_agg(h_src, ei, n_dst, w=None):
    msgs = h_src[ei[0]]
    if w is not None:
        msgs = msgs * w[:, None]
    s = jnp.zeros((n_dst, h_src.shape[1]), h_src.dtype).at[ei[1]].add(msgs)
    cnt = jnp.zeros((n_dst,), h_src.dtype).at[ei[1]].add(1.0)
    return s / jnp.maximum(cnt, 1.0)[:, None]


def _embed1(xp, xa, ei_pp, ei_aa, p, cnt_pp, cnt_aa):
    hp = xp @ p['W_in_p']
    ha = xa @ p['W_in_a']
    for l in range(2):
        sl = str(l)
        agg_p = _mean_agg_c(hp, ei_pp, _N, cnt_pp) @ p['W_pp_' + sl]
        agg_a = _mean_agg_c(ha, ei_aa, _N, cnt_aa) @ p['W_aa_' + sl]
        hp_n = jax.nn.relu(hp @ p['W_self_p_' + sl] + agg_p)
        ha_n = jax.nn.relu(ha @ p['W_self_a_' + sl] + agg_a)
        hp, ha = hp_n, ha_n
    return hp, ha


def _edge_pred0(h, ei, W1, b1, W2, b2):
    e = jnp.concatenate([h[ei[0]], h[ei[1]]], axis=1)
    return (jax.nn.relu(e @ W1 + b1) @ W2 + b2)[:, 0]


def _select(ei, pred0, seed):
    r, c = ei[0], ei[1]
    g = _gumbel_at(seed, r * _N + c)
    D, G = _sc_scatter_dg(r, c, pred0, g)
    S = _tc_topk(D, G)
    return _sc_member(r, c, S)


def kernel(x_paper, x_author, ei_pp, ei_aa, ei_pa, ei_ap, batch_paper,
           batch_author, index, params):
    p = params
    inv4, cnt4 = _sc_counts(ei_pp[1], ei_aa[1], ei_pa[1], ei_ap[1])
    hp1, ha1 = _embed1(x_paper, x_author, ei_pp, ei_aa, p, cnt4[0], cnt4[1])
    pred_pp = _edge_pred0(hp1, ei_pp, p['ep_pp_W1'], p['ep_pp_b1'],
                          p['ep_pp_W2'], p['ep_pp_b2'])
    pred_aa = _edge_pred0(ha1, ei_aa, p['ep_aa_W1'], p['ep_aa_b1'],
                          p['ep_aa_W2'], p['ep_aa_b2'])
    w_pp = _select(ei_pp, pred_pp, 42)
    w_aa = _select(ei_aa, pred_aa, 43)

    inv_pp = inv4[0].reshape(_N, 1)
    inv_aa = inv4[1].reshape(_N, 1)
    inv_pa = inv4[2].reshape(_N, 1)
    inv_ap = inv4[3].reshape(_N, 1)
    hp = _fused_matmul([(x_paper, p['W_in_p'], None)], relu=False)
    ha = _fused_matmul([(x_author, p['W_in_a'], None)], relu=False)
    for l in range(2):
        sl = str(l)
        hp_lo, hp_hi = hp[:, :128], hp[:, 128:]
        ha_lo, ha_hi = ha[:, :128], ha[:, 128:]
        s_pp = _sc_agg(hp_lo, hp_hi, ei_pp[0], ei_pp[1], w_pp)
        s_aa = _sc_agg(ha_lo, ha_hi, ei_aa[0], ei_aa[1], w_aa)
        s_ap = _sc_agg(ha_lo, ha_hi, ei_ap[0], ei_ap[1])
        s_pa = _sc_agg(hp_lo, hp_hi, ei_pa[0], ei_pa[1])
        W_pp, W_aa = p['W_pp_' + sl], p['W_aa_' + sl]
        W_ap, W_pa = p['W_ap_' + sl], p['W_pa_' + sl]
        hp_n = _fused_matmul(
            [(hp, p['W_self_p_' + sl], None),
             (s_pp[0], W_pp[:128], inv_pp), (s_pp[1], W_pp[128:], inv_pp),
             (s_ap[0], W_ap[:128], inv_ap), (s_ap[1], W_ap[128:], inv_ap)],
            relu=True)
        ha_n = _fused_matmul(
            [(ha, p['W_self_a_' + sl], None),
             (s_aa[0], W_aa[:128], inv_aa), (s_aa[1], W_aa[128:], inv_aa),
             (s_pa[0], W_pa[:128], inv_pa), (s_pa[1], W_pa[128:], inv_pa)],
            relu=True)
        hp, ha = hp_n, ha_n

    pool = hp[index].reshape(1, -1)
    y_hat = pool @ p['Wc'] + p['bc']
    return (y_hat, w_pp, w_aa)
